# baseline (speedup 1.0000x reference)
import functools

import jax
import jax.numpy as jnp
from jax import lax
from jax.experimental import pallas as pl
from jax.experimental.pallas import tpu as pltpu
from jax.experimental.pallas import tpu_sc as plsc

TOPK = 8
NUM_CORES = 2
NUM_SUBCORES = 16
NUM_WORKERS = NUM_CORES * NUM_SUBCORES
LANES = 16


def _router_block(x_ref, w_ref, b_ref, o_ref):
    logits = lax.dot_general(
        w_ref[...], x_ref[...],
        (((1,), (1,)), ((), ())),
        preferred_element_type=jnp.float32,
    ) + b_ref[...]
    m = jnp.max(logits, axis=0, keepdims=True)
    e = jnp.exp(logits - m)
    o_ref[...] = e / jnp.sum(e, axis=0, keepdims=True)


@functools.partial(jax.jit, static_argnames=("row0", "nrows", "block_rows"))
def _router_tc(x, W, b, row0=0, nrows=None, block_rows=1024):
    n, d = x.shape
    if nrows is None:
        nrows = n
    n_exp = W.shape[0]
    off = row0 // block_rows
    return pl.pallas_call(
        _router_block,
        grid=(nrows // block_rows,),
        in_specs=[
            pl.BlockSpec((block_rows, d), lambda i: (i + off, 0)),
            pl.BlockSpec((n_exp, d), lambda i: (0, 0)),
            pl.BlockSpec((n_exp, 1), lambda i: (0, 0)),
        ],
        out_specs=pl.BlockSpec((n_exp, block_rows), lambda i: (0, i)),
        out_shape=jax.ShapeDtypeStruct((n_exp, nrows), jnp.float32),
    )(x, W, b.reshape(n_exp, 1))


def _argmax_tree(vals):
    pairs = [(v, jnp.full((LANES,), e, jnp.int32)) for v, e in vals]
    while len(pairs) > 1:
        nxt = []
        for i in range(0, len(pairs), 2):
            (av, ai), (bv, bi) = pairs[i], pairs[i + 1]
            gt = bv > av
            nxt.append((jnp.where(gt, bv, av), jnp.where(gt, bi, ai)))
        pairs = nxt
    return pairs[0]


def _sc_gating(probs_t, n, n_exp):
    rpw = n // NUM_WORKERS
    groups = rpw // LANES
    slab = rpw * n_exp
    mesh = plsc.VectorSubcoreMesh(
        core_axis_name="c", subcore_axis_name="s")

    @functools.partial(
        pl.kernel,
        out_type=jax.ShapeDtypeStruct((n * n_exp,), jnp.float32),
        mesh=mesh,
        compiler_params=pltpu.CompilerParams(needs_layout_passes=False),
        scratch_types=[
            pltpu.VMEM((n_exp, rpw), jnp.float32),
            pltpu.VMEM((slab,), jnp.float32),
        ],
    )
    def gate(probs_hbm, out_hbm, pv, ov):
        wid = lax.axis_index("s") * NUM_CORES + lax.axis_index("c")
        pltpu.sync_copy(probs_hbm.at[:, pl.ds(wid * rpw, rpw)], pv)

        lane = lax.iota(jnp.int32, LANES)
        zero = jnp.zeros((LANES,), jnp.float32)
        neg = jnp.full((LANES,), -1.0, jnp.float32)

        def group_body(g, carry):
            col = g * LANES
            colv = col + lane
            obase = g * (LANES * n_exp) + lane * n_exp
            for j in range(n_exp):
                ov[pl.ds(g * (LANES * n_exp) + j * LANES, LANES)] = zero
            for _ in range(TOPK):
                vals = [(pv[e, pl.ds(col, LANES)], e)
                        for e in range(n_exp)]
                m, midx = _argmax_tree(vals)
                plsc.store_scatter(pv, [midx, colv], neg)
                plsc.store_scatter(ov, [obase + midx], m)
            return carry

        lax.fori_loop(0, groups, group_body, 0)
        pltpu.sync_copy(ov, out_hbm.at[pl.ds(wid * slab, slab)])

    return gate(probs_t)


def kernel(x, W, b):
    n, _ = x.shape
    n_exp = W.shape[0]
    chunks = 4
    nc = n // chunks
    outs = []
    for c in range(chunks):
        probs_t = _router_tc(x, W, b, row0=c * nc, nrows=nc)
        out_flat = _sc_gating(probs_t, nc, n_exp)
        outs.append(out_flat.reshape(nc, n_exp))
    return jnp.concatenate(outs, axis=0)


# speedup vs baseline: 1.0219x; 1.0219x over previous
import functools

import jax
import jax.numpy as jnp
from jax import lax
from jax.experimental import pallas as pl
from jax.experimental.pallas import tpu as pltpu
from jax.experimental.pallas import tpu_sc as plsc

TOPK = 8
NUM_CORES = 2
NUM_SUBCORES = 16
NUM_WORKERS = NUM_CORES * NUM_SUBCORES
LANES = 16


def _router_block(x_ref, w_ref, b_ref, o_ref):
    logits = lax.dot_general(
        w_ref[...], x_ref[...],
        (((1,), (1,)), ((), ())),
        preferred_element_type=jnp.float32,
    ) + b_ref[...]
    m = jnp.max(logits, axis=0, keepdims=True)
    e = jnp.exp(logits - m)
    o_ref[...] = e / jnp.sum(e, axis=0, keepdims=True)


@functools.partial(jax.jit, static_argnames=("row0", "nrows", "block_rows"))
def _router_tc(x, W, b, row0=0, nrows=None, block_rows=512):
    n, d = x.shape
    if nrows is None:
        nrows = n
    n_exp = W.shape[0]
    off = row0 // block_rows
    return pl.pallas_call(
        _router_block,
        grid=(nrows // block_rows,),
        in_specs=[
            pl.BlockSpec((block_rows, d), lambda i: (i + off, 0)),
            pl.BlockSpec((n_exp, d), lambda i: (0, 0)),
            pl.BlockSpec((n_exp, 1), lambda i: (0, 0)),
        ],
        out_specs=pl.BlockSpec((n_exp, block_rows), lambda i: (0, i)),
        out_shape=jax.ShapeDtypeStruct((n_exp, nrows), jnp.float32),
    )(x, W, b.reshape(n_exp, 1))


def _argmax_tree(vals):
    pairs = [(v, jnp.full((LANES,), e, jnp.int32)) for v, e in vals]
    while len(pairs) > 1:
        nxt = []
        for i in range(0, len(pairs), 2):
            (av, ai), (bv, bi) = pairs[i], pairs[i + 1]
            gt = bv > av
            nxt.append((jnp.where(gt, bv, av), jnp.where(gt, bi, ai)))
        pairs = nxt
    return pairs[0]


def _sc_gating(probs_t, n, n_exp):
    rpw = n // NUM_WORKERS
    groups = rpw // LANES
    slab = rpw * n_exp
    mesh = plsc.VectorSubcoreMesh(
        core_axis_name="c", subcore_axis_name="s")

    @functools.partial(
        pl.kernel,
        out_type=jax.ShapeDtypeStruct((n * n_exp,), jnp.float32),
        mesh=mesh,
        compiler_params=pltpu.CompilerParams(needs_layout_passes=False),
        scratch_types=[
            pltpu.VMEM((n_exp, rpw), jnp.float32),
            pltpu.VMEM((slab,), jnp.float32),
        ],
    )
    def gate(probs_hbm, out_hbm, pv, ov):
        wid = lax.axis_index("s") * NUM_CORES + lax.axis_index("c")
        pltpu.sync_copy(probs_hbm.at[:, pl.ds(wid * rpw, rpw)], pv)

        lane = lax.iota(jnp.int32, LANES)
        zero = jnp.zeros((LANES,), jnp.float32)
        neg = jnp.full((LANES,), -1.0, jnp.float32)

        def group_body(g, carry):
            col = g * LANES
            colv = col + lane
            obase = g * (LANES * n_exp) + lane * n_exp
            for j in range(n_exp):
                ov[pl.ds(g * (LANES * n_exp) + j * LANES, LANES)] = zero
            for _ in range(TOPK):
                vals = [(pv[e, pl.ds(col, LANES)], e)
                        for e in range(n_exp)]
                m, midx = _argmax_tree(vals)
                plsc.store_scatter(pv, [midx, colv], neg)
                plsc.store_scatter(ov, [obase + midx], m)
            return carry

        lax.fori_loop(0, groups, group_body, 0)
        pltpu.sync_copy(ov, out_hbm.at[pl.ds(wid * slab, slab)])

    return gate(probs_t)


def kernel(x, W, b):
    n, _ = x.shape
    n_exp = W.shape[0]
    chunks = 4
    nc = n // chunks
    outs = []
    for c in range(chunks):
        probs_t = _router_tc(x, W, b, row0=c * nc, nrows=nc)
        out_flat = _sc_gating(probs_t, nc, n_exp)
        outs.append(out_flat)
    return jnp.concatenate(outs, axis=0).reshape(n, n_exp)
